# baseline (speedup 1.0000x reference)
import functools

import jax
import jax.numpy as jnp
from jax import lax
from jax.experimental import pallas as pl
from jax.experimental.pallas import tpu as pltpu
from jax.experimental.pallas import tpu_sc as plsc

B, T, D = 4096, 200, 64
R = B * T
NC, NS = 2, 16
NW = NC * NS
ROWS_PER_W = R // NW
C = 400
NCHUNK = ROWS_PER_W // C
SUB = 80
LANES = 16


def _make_kernel():
  mesh = plsc.VectorSubcoreMesh(
      core_axis_name="c", subcore_axis_name="s",
      num_cores=NC, num_subcores=NS)

  @functools.partial(
      pl.kernel,
      out_type=jax.ShapeDtypeStruct((R, D), jnp.float32),
      mesh=mesh,
      scratch_types=[
          pltpu.VMEM((T, D), jnp.float32),
          pltpu.VMEM((C,), jnp.int32),
          pltpu.VMEM((C, D), jnp.float32),
          pltpu.SemaphoreType.DMA,
      ],
      compiler_params=pltpu.CompilerParams(use_tc_tiling_on_sc=False),
  )
  def k(x_hbm, emb_hbm, pos_hbm, out_hbm, pos_v, idx_v, rows_v, sem):
    wid = lax.axis_index("s") * NC + lax.axis_index("c")
    w_base = wid * ROWS_PER_W
    pltpu.sync_copy(pos_hbm, pos_v)

    def chunk_body(ci, carry):
      base = w_base + ci * C
      pltpu.sync_copy(x_hbm.at[pl.ds(base, C)], idx_v)
      copies = [
          pltpu.async_copy(
              emb_hbm.at[idx_v.at[pl.ds(j * SUB, SUB)]],
              rows_v.at[pl.ds(j * SUB, SUB)],
              sem,
          )
          for j in range(C // SUB)
      ]
      for cp in copies:
        cp.wait()

      def t_body(t, c2):
        for d in range(D // LANES):
          sl = pl.ds(d * LANES, LANES)
          pv = pos_v[t, sl]
          for s in range(C // T):
            r = s * T + t
            rows_v[r, sl] = rows_v[r, sl] + pv
        return c2

      lax.fori_loop(0, T, t_body, 0)
      pltpu.sync_copy(rows_v, out_hbm.at[pl.ds(base, C)])
      return carry

    lax.fori_loop(0, NCHUNK, chunk_body, 0)

  return k


_kernel = _make_kernel()


@jax.jit
def kernel(x, emb_table, pos_table):
  out = _kernel(x.reshape(R), emb_table, pos_table)
  return out.reshape(B, T, D)


# speedup vs baseline: 1.1215x; 1.1215x over previous
import functools

import jax
import jax.numpy as jnp
from jax import lax
from jax.experimental import pallas as pl
from jax.experimental.pallas import tpu as pltpu
from jax.experimental.pallas import tpu_sc as plsc

B, T, D = 4096, 200, 64
R = B * T
NC, NS = 2, 16
NW = NC * NS
ROWS_PER_W = R // NW
C = 400
NCHUNK = ROWS_PER_W // C
NSTEP = NCHUNK // 2
SUB = 80
LANES = 16


def _make_kernel():
  mesh = plsc.VectorSubcoreMesh(
      core_axis_name="c", subcore_axis_name="s",
      num_cores=NC, num_subcores=NS)

  @functools.partial(
      pl.kernel,
      out_type=jax.ShapeDtypeStruct((R, D), jnp.float32),
      mesh=mesh,
      scratch_types=[
          pltpu.VMEM((T, D), jnp.float32),
          pltpu.VMEM((ROWS_PER_W,), jnp.int32),
          pltpu.VMEM((C, D), jnp.float32),
          pltpu.VMEM((C, D), jnp.float32),
          pltpu.SemaphoreType.DMA,
          pltpu.SemaphoreType.DMA,
          pltpu.SemaphoreType.DMA,
          pltpu.SemaphoreType.DMA,
      ],
      compiler_params=pltpu.CompilerParams(use_tc_tiling_on_sc=False),
  )
  def k(x_hbm, emb_hbm, pos_hbm, out_hbm, pos_v, idx_v,
        rows0, rows1, sem_g0, sem_g1, sem_s0, sem_s1):
    wid = lax.axis_index("s") * NC + lax.axis_index("c")
    w_base = wid * ROWS_PER_W
    rows = (rows0, rows1)
    sem_g = (sem_g0, sem_g1)
    sem_s = (sem_s0, sem_s1)

    pltpu.sync_copy(pos_hbm, pos_v)
    pltpu.sync_copy(x_hbm.at[pl.ds(w_base, ROWS_PER_W)], idx_v)

    def issue_gather(ci, slot):
      for j in range(C // SUB):
        pltpu.async_copy(
            emb_hbm.at[idx_v.at[pl.ds(ci * C + j * SUB, SUB)]],
            rows[slot].at[pl.ds(j * SUB, SUB)],
            sem_g[slot])

    def wait_gather(slot):
      pltpu.make_async_copy(
          emb_hbm.at[idx_v.at[pl.ds(0, C)]], rows[slot], sem_g[slot]).wait()

    def issue_store(ci, slot):
      pltpu.async_copy(
          rows[slot], out_hbm.at[pl.ds(w_base + ci * C, C)], sem_s[slot])

    def wait_store(slot):
      pltpu.make_async_copy(
          rows[slot], out_hbm.at[pl.ds(w_base, C)], sem_s[slot]).wait()

    def add_pos(slot):
      def t_body(t, c2):
        for d in range(D // LANES):
          sl = pl.ds(d * LANES, LANES)
          pv = pos_v[t, sl]
          for s in range(C // T):
            r = s * T + t
            rows[slot][r, sl] = rows[slot][r, sl] + pv
        return c2
      lax.fori_loop(0, T, t_body, 0)

    issue_gather(0, 0)

    def step(i, carry):
      c0 = 2 * i
      wait_gather(0)

      @pl.when(i > 0)
      def _():
        wait_store(1)
      issue_gather(c0 + 1, 1)
      add_pos(0)
      issue_store(c0, 0)

      wait_gather(1)

      @pl.when(i < NSTEP - 1)
      def _():
        wait_store(0)
        issue_gather(c0 + 2, 0)
      add_pos(1)
      issue_store(c0 + 1, 1)
      return carry

    lax.fori_loop(0, NSTEP, step, 0)
    wait_store(0)
    wait_store(1)

  return k


_kernel = _make_kernel()


@jax.jit
def kernel(x, emb_table, pos_table):
  out = _kernel(x.reshape(R), emb_table, pos_table)
  return out.reshape(B, T, D)
